# baseline (speedup 1.0000x reference)
import functools

import jax
import jax.numpy as jnp
from jax import lax
from jax.experimental import pallas as pl
from jax.experimental.pallas import tpu as pltpu
from jax.experimental.pallas import tpu_sc as plsc

B, DEC, ENC, V = 4, 64, 512, 32000
H, A = 1024, 192
R = B * DEC
L = 16
G = ENC // L
NW = 32
SPLIT = 128
NBUF = 3


def _sigmoid(x):
    return 1.0 / (1.0 + jnp.exp(-x))


def _dense_body(att_ref, hid_ref, wiv_ref, biv_ref, wpg_ref, bpg_ref,
                add_ref, pg_ref):
    att = att_ref[...]
    iva = jnp.sum(att * wiv_ref[...][None, :, :], axis=1)
    iva = _sigmoid(iva + biv_ref[0, 0])
    pg = _sigmoid(hid_ref[...] @ wpg_ref[...] + bpg_ref[0, 0])
    add_ref[...] = (1.0 - pg) * iva
    pg_ref[...] = jnp.broadcast_to(pg, (pg.shape[0], 128))


def _dense_chunk(row0, nrows, att3, hid2, W_iv, b_iv2, W_pg, b_pg2):
    RB = 8
    off = row0 // RB
    return pl.pallas_call(
        _dense_body,
        grid=(nrows // RB,),
        in_specs=[
            pl.BlockSpec((RB, A, ENC), lambda i: (i + off, 0, 0)),
            pl.BlockSpec((RB, H), lambda i: (i + off, 0)),
            pl.BlockSpec((A, 1), lambda i: (0, 0)),
            pl.BlockSpec((1, 1), lambda i: (0, 0)),
            pl.BlockSpec((H, 1), lambda i: (0, 0)),
            pl.BlockSpec((1, 1), lambda i: (0, 0)),
        ],
        out_specs=[
            pl.BlockSpec((RB, ENC), lambda i: (i, 0)),
            pl.BlockSpec((RB, 128), lambda i: (i, 0)),
        ],
        out_shape=[
            jax.ShapeDtypeStruct((nrows, ENC), jnp.float32),
            jax.ShapeDtypeStruct((nrows, 128), jnp.float32),
        ],
        name=f"dense_rows{row0}",
    )(att3, hid2, W_iv, b_iv2, W_pg, b_pg2)


def _make_sc_kernel(row0, nrows, makes_output):
    rows_w = nrows // NW
    mesh = plsc.VectorSubcoreMesh(core_axis_name="c", subcore_axis_name="s")
    scratch = (
        [pltpu.VMEM((V,), jnp.float32) for _ in range(NBUF)]
        + [
            pltpu.VMEM((rows_w * ENC,), jnp.float32),
            pltpu.VMEM((rows_w * 128,), jnp.float32),
            pltpu.VMEM((B * ENC,), jnp.int32),
        ]
        + [pltpu.SemaphoreType.DMA] * (2 * NBUF + 1)
    )

    @functools.partial(
        pl.kernel,
        mesh=mesh,
        out_type=(jax.ShapeDtypeStruct((R, V), jnp.float32)
                  if makes_output else ()),
        scratch_types=scratch,
        compiler_params=pltpu.CompilerParams(needs_layout_passes=False),
        name=f"sc_scatter_rows{row0}",
    )
    def sc_kernel(ovp_hbm, pg_hbm, add_hbm, idx_hbm, out_hbm, *scr):
        bufs = scr[:NBUF]
        add_v, pg_v, idx_v = scr[NBUF:NBUF + 3]
        in_sems = scr[NBUF + 3:NBUF + 3 + NBUF]
        out_sems = scr[NBUF + 3 + NBUF:NBUF + 3 + 2 * NBUF]
        pre_sem = scr[NBUF + 3 + 2 * NBUF]

        wid = lax.axis_index("s") * 2 + lax.axis_index("c")
        base = wid * rows_w

        pres = []
        for i in range(rows_w):
            pres.append(pltpu.async_copy(
                add_hbm.at[base + i], add_v.at[pl.ds(i * ENC, ENC)], pre_sem))
            pres.append(pltpu.async_copy(
                pg_hbm.at[base + i], pg_v.at[pl.ds(i * 128, 128)], pre_sem))
        pres.append(pltpu.async_copy(idx_hbm, idx_v, pre_sem))

        in_descs = [None] * rows_w
        out_descs = [None] * rows_w

        def start_in(r):
            in_descs[r] = pltpu.async_copy(
                ovp_hbm.at[row0 + base + r], bufs[r % NBUF], in_sems[r % NBUF])

        for r in range(min(NBUF, rows_w)):
            start_in(r)

        for p in pres:
            p.wait()

        for i in range(rows_w):
            row = bufs[i % NBUF]
            in_descs[i].wait()

            pgv = pg_v[pl.ds(i * 128, L)]

            UN = 8

            def scale_body(j, carry):
                off = j * (L * UN)
                for k in range(UN):
                    o = off + k * L
                    row[pl.ds(o, L)] = row[pl.ds(o, L)] * pgv
                return carry

            lax.fori_loop(0, V // (L * UN), scale_body, 0)

            b_i = (row0 + base + i) // DEC

            def scatter_group(g, carry):
                e0 = g * L
                il = idx_v[pl.ds(b_i * ENC + e0, L)]
                v = add_v[pl.ds(i * ENC + e0, L)]
                plsc.addupdate_scatter(row, [il], v)
                return carry

            lax.fori_loop(0, G, scatter_group, 0)

            out_descs[i] = pltpu.async_copy(
                row, out_hbm.at[row0 + base + i], out_sems[i % NBUF])
            if i + NBUF < rows_w:
                out_descs[i].wait()
                start_in(i + NBUF)

        for i in range(max(0, rows_w - NBUF), rows_w):
            out_descs[i].wait()

    return sc_kernel


_sc_kernel0 = _make_sc_kernel(0, SPLIT, True)
_sc_kernel1 = _make_sc_kernel(SPLIT, R - SPLIT, False)


def kernel(input_ids, attentions, hidden_states, output_vocabulary_probabilities,
           W_pg, b_pg, W_iv, b_iv):
    att3 = attentions.transpose(0, 1, 3, 2).reshape(R, A, ENC)
    hid2 = hidden_states.reshape(R, H)
    ovp2 = output_vocabulary_probabilities.reshape(R, V)
    b_iv2 = b_iv.reshape(1, 1)
    b_pg2 = b_pg.reshape(1, 1)

    add0, pg0 = _dense_chunk(0, SPLIT, att3, hid2, W_iv, b_iv2, W_pg, b_pg2)
    ids_flat = input_ids.reshape(B * ENC)
    out_full = _sc_kernel0(ovp2, pg0, add0, ids_flat)
    out_ref = jax.new_ref(out_full)
    add1, pg1 = _dense_chunk(SPLIT, R - SPLIT, att3, hid2, W_iv, b_iv2,
                             W_pg, b_pg2)
    _sc_kernel1(ovp2, pg1, add1, ids_flat, out_ref)
    return jax.freeze(out_ref).reshape(B, DEC, V)
